# baseline (speedup 1.0000x reference)
import jax
from jax.experimental import pallas as pl
from jax.experimental.pallas import tpu as pltpu


def _copy_body(x_ref, o_ref):
    o_ref[...] = x_ref[...]


def kernel(x):
    n = x.shape[0]
    return pl.pallas_call(
        _copy_body,
        grid=(2,),
        in_specs=[pl.BlockSpec((n // 2, x.shape[1]), lambda i: (i, 0))],
        out_specs=pl.BlockSpec((n // 2, x.shape[1]), lambda i: (i, 0)),
        out_shape=jax.ShapeDtypeStruct(x.shape, x.dtype),
        compiler_params=pltpu.CompilerParams(dimension_semantics=("arbitrary",), skip_device_barrier=True),
    )(x)


# speedup vs baseline: 1.0084x; 1.0084x over previous
import jax
from jax.experimental import pallas as pl
from jax.experimental.pallas import tpu as pltpu


def _copy_body(x_ref, o_ref):
    o_ref[...] = x_ref[...]


def kernel(x):
    n = x.shape[0]
    return pl.pallas_call(
        _copy_body,
        grid=(2,),
        in_specs=[pl.BlockSpec((n // 2, x.shape[1]), lambda i: (i, 0))],
        out_specs=pl.BlockSpec((n // 2, x.shape[1]), lambda i: (i, 0)),
        out_shape=jax.ShapeDtypeStruct(x.shape, x.dtype),
    )(x)
